# baseline (speedup 1.0000x reference)
import functools

import jax
import jax.numpy as jnp
import numpy as np
from jax import lax
from jax.experimental import pallas as pl
from jax.experimental.pallas import tpu as pltpu
from jax.experimental.pallas import tpu_sc as plsc

B, S, H, V, HCM, SL, L, I = 2, 2048, 768, 8192, 256, 32, 2, 3072

TS = 512
LT_R = 1024
LT_V = 2048


def _decay_weights():
    t = np.arange(S)
    n_flags = ((S - 1) // 10) + 1
    m = np.ceil(t / 10).astype(np.int64)
    pow9 = np.ones(n_flags + 1, dtype=np.float64)
    for k in range(1, n_flags + 1):
        pow9[k] = pow9[k - 1] * 0.9
    w = pow9[n_flags - m].astype(np.float32)
    return jnp.asarray(w.reshape(S // TS, 1, TS))


def _sc_embed_gather(table, idx_flat):
    info = plsc.get_sparse_core_info()
    nc, ns = info.num_cores, info.num_subcores
    nw = nc * ns
    rows = idx_flat.shape[0]
    bpw = rows // nw
    mesh = plsc.VectorSubcoreMesh(core_axis_name="c", subcore_axis_name="s")

    @functools.partial(
        pl.kernel,
        mesh=mesh,
        out_type=jax.ShapeDtypeStruct((rows, H), jnp.float32),
        scratch_types=[
            pltpu.VMEM((bpw,), jnp.int32),
            pltpu.VMEM((bpw, H), jnp.float32),
            pltpu.SemaphoreType.DMA,
        ],
    )
    def gather_k(table_hbm, idx_hbm, out_hbm, idx_v, rows_v, sem):
        wid = lax.axis_index("s") * nc + lax.axis_index("c")
        base = wid * bpw
        pltpu.sync_copy(idx_hbm.at[pl.ds(base, bpw)], idx_v)
        pltpu.async_copy(table_hbm.at[idx_v], rows_v, sem).wait()
        pltpu.sync_copy(rows_v, out_hbm.at[pl.ds(base, bpw)])

    return gather_k(table, idx_flat)


def _ln(x, g, b):
    m = x.mean(-1, keepdims=True)
    v = ((x - m) ** 2).mean(-1, keepdims=True)
    return (x - m) / jnp.sqrt(v + 1e-5) * g + b


def _dot(a, b):
    return jnp.dot(a, b, preferred_element_type=jnp.float32)


def _make_layer_body(add_pos, final_ln):
    def body(*refs):
        it = iter(refs)
        h_ref = next(it)
        pos_ref = next(it) if add_pos else None
        w_ref = next(it)
        wi, bi = next(it), next(it)
        wg, bg = next(it), next(it)
        wq, bq = next(it), next(it)
        wo, bo = next(it), next(it)
        ln_g, ln_b = next(it), next(it)
        f1, fb1 = next(it), next(it)
        f2, fb2 = next(it), next(it)
        fln_g, fln_b = next(it), next(it)
        if final_ln:
            fin_g, fin_b = next(it), next(it)
        h_out = next(it)
        fast_out = next(it)
        slow_out = next(it)
        hln_out = next(it) if final_ln else None

        t = pl.program_id(1)
        h = h_ref[0]
        if add_pos:
            h = h + pos_ref[...]

        items = _dot(h, wi[...]) + bi[...]
        fg = jax.nn.sigmoid(_dot(h, wg[...]) + bg[...])
        wv = w_ref[0]
        wfg = fg * wv.reshape(TS, 1)
        dn = (((0,), (0,)), ((), ()))
        fa = lax.dot_general(wfg, items, dn,
                             preferred_element_type=jnp.float32)
        sa = lax.dot_general(fg - wfg, items, dn,
                             preferred_element_type=jnp.float32)

        @pl.when(t == 0)
        def _():
            fast_out[0] = fa
            slow_out[0] = sa

        @pl.when(t != 0)
        def _():
            fast_out[0] += fa
            slow_out[0] += sa

        query = _dot(h, wq[...]) + bq[...]
        out = _dot(query, wo[...]) + bo[...]
        h1 = _ln(h + out, ln_g[...], ln_b[...])
        act = jax.nn.gelu(_dot(h1.astype(jnp.bfloat16), f1[...]) + fb1[...])
        ffn = _dot(act.astype(jnp.bfloat16), f2[...]) + fb2[...]
        h2 = _ln(h1 + ffn, fln_g[...], fln_b[...])
        h_out[0] = h2
        if final_ln:
            hln_out[0] = _ln(h2, fin_g[...], fin_b[...])

    return body


def _layer_call(h, p, w3, pos=None, final=None):
    add_pos = pos is not None
    final_ln = final is not None
    row = lambda x: x.reshape(1, -1)
    full2 = lambda a: pl.BlockSpec(a.shape, lambda b, t: (0, 0))

    inputs = [h]
    in_specs = [pl.BlockSpec((1, TS, H), lambda b, t: (b, t, 0))]
    if add_pos:
        inputs.append(pos)
        in_specs.append(pl.BlockSpec((TS, H), lambda b, t: (t, 0)))
    inputs.append(w3)
    in_specs.append(pl.BlockSpec((1, 1, TS), lambda b, t: (t, 0, 0)))

    wmats = [
        p['Wi'], row(p['bi']),
        p['Wg'][:H, :SL], row(p['bg'][:SL]),
        p['Wq'], row(p['bq']),
        p['Wo'][:H], row(p['bo']),
        row(p['ln_g']), row(p['ln_b']),
        p['F1'].astype(jnp.bfloat16), row(p['fb1']),
        p['F2'].astype(jnp.bfloat16), row(p['fb2']),
        row(p['fln_g']), row(p['fln_b']),
    ]
    if final_ln:
        wmats += [row(final[0]), row(final[1])]
    inputs += wmats
    in_specs += [full2(a) for a in wmats]

    out_shape = [
        jax.ShapeDtypeStruct((B, S, H), jnp.float32),
        jax.ShapeDtypeStruct((B, SL, HCM), jnp.float32),
        jax.ShapeDtypeStruct((B, SL, HCM), jnp.float32),
    ]
    out_specs = [
        pl.BlockSpec((1, TS, H), lambda b, t: (b, t, 0)),
        pl.BlockSpec((1, SL, HCM), lambda b, t: (b, 0, 0)),
        pl.BlockSpec((1, SL, HCM), lambda b, t: (b, 0, 0)),
    ]
    if final_ln:
        out_shape.append(jax.ShapeDtypeStruct((B, S, H), jnp.float32))
        out_specs.append(pl.BlockSpec((1, TS, H), lambda b, t: (b, t, 0)))

    return pl.pallas_call(
        _make_layer_body(add_pos, final_ln),
        grid=(B, S // TS),
        in_specs=in_specs,
        out_specs=out_specs,
        out_shape=out_shape,
    )(*inputs)


def _logits_body(h_ref, emb_ref, out_ref):
    out_ref[...] = lax.dot_general(
        h_ref[...].astype(jnp.bfloat16), emb_ref[...], (((1,), (1,)), ((), ())),
        preferred_element_type=jnp.float32)


def _logits_call(hln_flat, tok_emb):
    rows = hln_flat.shape[0]
    return pl.pallas_call(
        _logits_body,
        grid=(V // LT_V, rows // LT_R),
        in_specs=[
            pl.BlockSpec((LT_R, H), lambda v, r: (r, 0)),
            pl.BlockSpec((LT_V, H), lambda v, r: (v, 0)),
        ],
        out_specs=pl.BlockSpec((LT_R, LT_V), lambda v, r: (r, v)),
        out_shape=jax.ShapeDtypeStruct((rows, V), jnp.float32),
    )(hln_flat, tok_emb)


def kernel(input_ids, params):
    ids_flat = input_ids.reshape(-1).astype(jnp.int32)
    emb = _sc_embed_gather(params['tok_emb'], ids_flat)
    h = emb.reshape(B, S, H)
    w3 = _decay_weights()
    pos = params['pos_emb'][:S]

    layers = params['layers']
    fasts, slows = [], []
    for li, p in enumerate(layers):
        pos_arg = pos if li == 0 else None
        final = (params['final_g'], params['final_b']) if li == L - 1 else None
        res = _layer_call(h, p, w3, pos=pos_arg, final=final)
        if final is not None:
            h, fast, slow, hln = res
        else:
            h, fast, slow = res
        fasts.append(fast)
        slows.append(slow)

    logits = _logits_call(hln.reshape(B * S, H),
                          params['tok_emb'].astype(jnp.bfloat16))
    return logits.reshape(B, S, V), jnp.stack(fasts), jnp.stack(slows)


# speedup vs baseline: 1.0705x; 1.0705x over previous
import functools

import jax
import jax.numpy as jnp
import numpy as np
from jax import lax
from jax.experimental import pallas as pl
from jax.experimental.pallas import tpu as pltpu
from jax.experimental.pallas import tpu_sc as plsc

B, S, H, V, HCM, SL, L, I = 2, 2048, 768, 8192, 256, 32, 2, 3072

TS = 256
LT_R = 1024
LT_V = 2048


def _decay_weights():
    t = np.arange(S)
    n_flags = ((S - 1) // 10) + 1
    m = np.ceil(t / 10).astype(np.int64)
    pow9 = np.ones(n_flags + 1, dtype=np.float64)
    for k in range(1, n_flags + 1):
        pow9[k] = pow9[k - 1] * 0.9
    w = pow9[n_flags - m].astype(np.float32)
    return jnp.asarray(w.reshape(S // TS, 1, TS))


def _sc_embed_gather(table, idx_flat):
    info = plsc.get_sparse_core_info()
    nc, ns = info.num_cores, info.num_subcores
    nw = nc * ns
    rows = idx_flat.shape[0]
    bpw = rows // nw
    mesh = plsc.VectorSubcoreMesh(core_axis_name="c", subcore_axis_name="s")

    @functools.partial(
        pl.kernel,
        mesh=mesh,
        out_type=jax.ShapeDtypeStruct((rows, H), jnp.float32),
        scratch_types=[
            pltpu.VMEM((bpw,), jnp.int32),
            pltpu.VMEM((bpw, H), jnp.float32),
            pltpu.SemaphoreType.DMA,
        ],
    )
    def gather_k(table_hbm, idx_hbm, out_hbm, idx_v, rows_v, sem):
        wid = lax.axis_index("s") * nc + lax.axis_index("c")
        base = wid * bpw
        pltpu.sync_copy(idx_hbm.at[pl.ds(base, bpw)], idx_v)
        pltpu.async_copy(table_hbm.at[idx_v], rows_v, sem).wait()
        pltpu.sync_copy(rows_v, out_hbm.at[pl.ds(base, bpw)])

    return gather_k(table, idx_flat)


def _ln(x, g, b):
    m = x.mean(-1, keepdims=True)
    v = ((x - m) ** 2).mean(-1, keepdims=True)
    return (x - m) / jnp.sqrt(v + 1e-5) * g + b


def _dot(a, b):
    return jnp.dot(a, b, preferred_element_type=jnp.float32)


N_LW = 16


def _layers_body(*refs):
    it = iter(refs)
    h_ref = next(it)
    pos_ref = next(it)
    w_ref = next(it)
    lw = [[next(it) for _ in range(N_LW)] for _ in range(L)]
    fin_g, fin_b = next(it), next(it)
    hln_out = next(it)
    facc = [next(it) for _ in range(L)]
    sacc = [next(it) for _ in range(L)]

    t = pl.program_id(1)
    h = h_ref[0] + pos_ref[...]
    wv = w_ref[0].reshape(TS, 1)
    dn = (((0,), (0,)), ((), ()))

    for li in range(L):
        (wi, bi, wg, bg, wq, bq, wo, bo, ln_g, ln_b,
         f1, fb1, f2, fb2, fln_g, fln_b) = lw[li]
        items = _dot(h, wi[...]) + bi[...]
        fg = jax.nn.sigmoid(_dot(h, wg[...]) + bg[...])
        wfg = fg * wv
        fa = lax.dot_general(wfg, items, dn,
                             preferred_element_type=jnp.float32)
        sa = lax.dot_general(fg - wfg, items, dn,
                             preferred_element_type=jnp.float32)

        @pl.when(t == 0)
        def _(li=li, fa=fa, sa=sa):
            facc[li][0] = fa
            sacc[li][0] = sa

        @pl.when(t != 0)
        def _(li=li, fa=fa, sa=sa):
            facc[li][0] += fa
            sacc[li][0] += sa

        query = _dot(h, wq[...]) + bq[...]
        out = _dot(query, wo[...]) + bo[...]
        h1 = _ln(h + out, ln_g[...], ln_b[...])
        ffn = _dot(jax.nn.gelu(_dot(h1, f1[...]) + fb1[...]), f2[...]) + fb2[...]
        h = _ln(h1 + ffn, fln_g[...], fln_b[...])

    hln_out[0] = _ln(h, fin_g[...], fin_b[...])


def _layers_call(h, params, w3, pos):
    row = lambda x: x.reshape(1, -1)
    full2 = lambda a: pl.BlockSpec(a.shape, lambda b, t: (0, 0))

    inputs = [h, pos, w3]
    in_specs = [
        pl.BlockSpec((1, TS, H), lambda b, t: (b, t, 0)),
        pl.BlockSpec((TS, H), lambda b, t: (t, 0)),
        pl.BlockSpec((1, 1, TS), lambda b, t: (t, 0, 0)),
    ]
    for p in params['layers']:
        wmats = [
            p['Wi'], row(p['bi']),
            p['Wg'][:H, :SL], row(p['bg'][:SL]),
            p['Wq'], row(p['bq']),
            p['Wo'][:H], row(p['bo']),
            row(p['ln_g']), row(p['ln_b']),
            p['F1'], row(p['fb1']),
            p['F2'], row(p['fb2']),
            row(p['fln_g']), row(p['fln_b']),
        ]
        inputs += wmats
        in_specs += [full2(a) for a in wmats]
    fin = [row(params['final_g']), row(params['final_b'])]
    inputs += fin
    in_specs += [full2(a) for a in fin]

    acc_spec = pl.BlockSpec((1, SL, HCM), lambda b, t: (b, 0, 0))
    acc_shape = jax.ShapeDtypeStruct((B, SL, HCM), jnp.float32)
    out_shape = [jax.ShapeDtypeStruct((B, S, H), jnp.float32)]
    out_specs = [pl.BlockSpec((1, TS, H), lambda b, t: (b, t, 0))]
    out_shape += [acc_shape] * (2 * L)
    out_specs += [acc_spec] * (2 * L)

    return pl.pallas_call(
        _layers_body,
        grid=(B, S // TS),
        in_specs=in_specs,
        out_specs=out_specs,
        out_shape=out_shape,
    )(*inputs)


def _logits_body(h_ref, emb_ref, out_ref):
    out_ref[...] = lax.dot_general(
        h_ref[...], emb_ref[...], (((1,), (1,)), ((), ())),
        preferred_element_type=jnp.float32)


def _logits_call(hln_flat, tok_emb):
    rows = hln_flat.shape[0]
    return pl.pallas_call(
        _logits_body,
        grid=(V // LT_V, rows // LT_R),
        in_specs=[
            pl.BlockSpec((LT_R, H), lambda v, r: (r, 0)),
            pl.BlockSpec((LT_V, H), lambda v, r: (v, 0)),
        ],
        out_specs=pl.BlockSpec((LT_R, LT_V), lambda v, r: (r, v)),
        out_shape=jax.ShapeDtypeStruct((rows, V), jnp.float32),
    )(hln_flat, tok_emb)


def kernel(input_ids, params):
    ids_flat = input_ids.reshape(-1).astype(jnp.int32)
    emb = _sc_embed_gather(params['tok_emb'], ids_flat)
    h = emb.reshape(B, S, H)
    w3 = _decay_weights()
    pos = params['pos_emb'][:S]

    res = _layers_call(h, params, w3, pos)
    hln = res[0]
    fasts = jnp.stack(res[1:1 + L])
    slows = jnp.stack(res[1 + L:1 + 2 * L])

    logits = _logits_call(hln.reshape(B * S, H), params['tok_emb'])
    return logits.reshape(B, S, V), fasts, slows


# speedup vs baseline: 1.1641x; 1.0875x over previous
import functools

import jax
import jax.numpy as jnp
import numpy as np
from jax import lax
from jax.experimental import pallas as pl
from jax.experimental.pallas import tpu as pltpu
from jax.experimental.pallas import tpu_sc as plsc

B, S, H, V, HCM, SL, L, I = 2, 2048, 768, 8192, 256, 32, 2, 3072

TS = 1024
LT_R = 1024
LT_V = 2048


def _decay_weights():
    t = np.arange(S)
    n_flags = ((S - 1) // 10) + 1
    m = np.ceil(t / 10).astype(np.int64)
    pow9 = np.ones(n_flags + 1, dtype=np.float64)
    for k in range(1, n_flags + 1):
        pow9[k] = pow9[k - 1] * 0.9
    w = pow9[n_flags - m].astype(np.float32)
    return jnp.asarray(w.reshape(S // TS, 1, TS))


def _sc_embed_gather(table, idx_flat):
    info = plsc.get_sparse_core_info()
    nc, ns = info.num_cores, info.num_subcores
    nw = nc * ns
    rows = idx_flat.shape[0]
    bpw = rows // nw
    mesh = plsc.VectorSubcoreMesh(core_axis_name="c", subcore_axis_name="s")

    @functools.partial(
        pl.kernel,
        mesh=mesh,
        out_type=jax.ShapeDtypeStruct((rows, H), jnp.float32),
        scratch_types=[
            pltpu.VMEM((bpw,), jnp.int32),
            pltpu.VMEM((bpw, H), jnp.float32),
            pltpu.SemaphoreType.DMA,
        ],
    )
    def gather_k(table_hbm, idx_hbm, out_hbm, idx_v, rows_v, sem):
        wid = lax.axis_index("s") * nc + lax.axis_index("c")
        base = wid * bpw
        pltpu.sync_copy(idx_hbm.at[pl.ds(base, bpw)], idx_v)
        pltpu.async_copy(table_hbm.at[idx_v], rows_v, sem).wait()
        pltpu.sync_copy(rows_v, out_hbm.at[pl.ds(base, bpw)])

    return gather_k(table, idx_flat)


def _ln(x, g, b):
    m = x.mean(-1, keepdims=True)
    v = ((x - m) ** 2).mean(-1, keepdims=True)
    return (x - m) / jnp.sqrt(v + 1e-5) * g + b


def _dot(a, b):
    return jnp.dot(a, b, preferred_element_type=jnp.float32)


def _make_layer_body(add_pos, final_ln):
    def body(*refs):
        it = iter(refs)
        h_ref = next(it)
        pos_ref = next(it) if add_pos else None
        w_ref = next(it)
        wi, bi = next(it), next(it)
        wg, bg = next(it), next(it)
        wq, bq = next(it), next(it)
        wo, bo = next(it), next(it)
        ln_g, ln_b = next(it), next(it)
        f1, fb1 = next(it), next(it)
        f2, fb2 = next(it), next(it)
        fln_g, fln_b = next(it), next(it)
        if final_ln:
            fin_g, fin_b = next(it), next(it)
        h_out = next(it)
        fast_out = next(it)
        slow_out = next(it)

        t = pl.program_id(1)
        h = h_ref[0]
        if add_pos:
            h = h + pos_ref[...]

        items = _dot(h, wi[...]) + bi[...]
        fg = jax.nn.sigmoid(_dot(h, wg[...]) + bg[...])
        wv = w_ref[0]
        wfg = fg * wv.reshape(TS, 1)
        dn = (((0,), (0,)), ((), ()))
        fa = lax.dot_general(wfg, items, dn,
                             preferred_element_type=jnp.float32)
        sa = lax.dot_general(fg - wfg, items, dn,
                             preferred_element_type=jnp.float32)

        @pl.when(t == 0)
        def _():
            fast_out[0] = fa
            slow_out[0] = sa

        @pl.when(t != 0)
        def _():
            fast_out[0] += fa
            slow_out[0] += sa

        query = _dot(h, wq[...]) + bq[...]
        out = _dot(query, wo[...]) + bo[...]
        h1 = _ln(h + out, ln_g[...], ln_b[...])
        ffn = fb2[...]
        half = I // 2
        for c in range(2):
            cs = pl.ds(c * half, half)
            act = jax.nn.gelu(_dot(h1, f1[:, cs]) + fb1[:, cs])
            ffn = ffn + _dot(act, f2[cs, :])
        h2 = _ln(h1 + ffn, fln_g[...], fln_b[...])
        if final_ln:
            h_out[0] = _ln(h2, fin_g[...], fin_b[...])
        else:
            h_out[0] = h2

    return body


def _layer_call(h, p, w3, pos=None, final=None):
    add_pos = pos is not None
    final_ln = final is not None
    row = lambda x: x.reshape(1, -1)
    full2 = lambda a: pl.BlockSpec(a.shape, lambda b, t: (0, 0))

    inputs = [h]
    in_specs = [pl.BlockSpec((1, TS, H), lambda b, t: (b, t, 0))]
    if add_pos:
        inputs.append(pos)
        in_specs.append(pl.BlockSpec((TS, H), lambda b, t: (t, 0)))
    inputs.append(w3)
    in_specs.append(pl.BlockSpec((1, 1, TS), lambda b, t: (t, 0, 0)))

    wmats = [
        p['Wi'], row(p['bi']),
        p['Wg'][:H, :SL], row(p['bg'][:SL]),
        p['Wq'], row(p['bq']),
        p['Wo'][:H], row(p['bo']),
        row(p['ln_g']), row(p['ln_b']),
        p['F1'], row(p['fb1']),
        p['F2'], row(p['fb2']),
        row(p['fln_g']), row(p['fln_b']),
    ]
    if final_ln:
        wmats += [row(final[0]), row(final[1])]
    inputs += wmats
    in_specs += [full2(a) for a in wmats]

    out_shape = [
        jax.ShapeDtypeStruct((B, S, H), jnp.float32),
        jax.ShapeDtypeStruct((B, SL, HCM), jnp.float32),
        jax.ShapeDtypeStruct((B, SL, HCM), jnp.float32),
    ]
    out_specs = [
        pl.BlockSpec((1, TS, H), lambda b, t: (b, t, 0)),
        pl.BlockSpec((1, SL, HCM), lambda b, t: (b, 0, 0)),
        pl.BlockSpec((1, SL, HCM), lambda b, t: (b, 0, 0)),
    ]

    return pl.pallas_call(
        _make_layer_body(add_pos, final_ln),
        grid=(B, S // TS),
        in_specs=in_specs,
        out_specs=out_specs,
        out_shape=out_shape,
    )(*inputs)


def _logits_body(h_ref, emb_ref, out_ref):
    out_ref[...] = lax.dot_general(
        h_ref[...], emb_ref[...], (((1,), (1,)), ((), ())),
        preferred_element_type=jnp.float32)


def _logits_call(hln_flat, tok_emb):
    rows = hln_flat.shape[0]
    return pl.pallas_call(
        _logits_body,
        grid=(V // LT_V, rows // LT_R),
        in_specs=[
            pl.BlockSpec((LT_R, H), lambda v, r: (r, 0)),
            pl.BlockSpec((LT_V, H), lambda v, r: (v, 0)),
        ],
        out_specs=pl.BlockSpec((LT_R, LT_V), lambda v, r: (r, v)),
        out_shape=jax.ShapeDtypeStruct((rows, V), jnp.float32),
    )(hln_flat, tok_emb)


def kernel(input_ids, params):
    ids_flat = input_ids.reshape(-1).astype(jnp.int32)
    emb = _sc_embed_gather(params['tok_emb'], ids_flat)
    h = emb.reshape(B, S, H)
    w3 = _decay_weights()
    pos = params['pos_emb'][:S]

    layers = params['layers']
    fasts, slows = [], []
    for li, p in enumerate(layers):
        pos_arg = pos if li == 0 else None
        final = (params['final_g'], params['final_b']) if li == L - 1 else None
        h, fast, slow = _layer_call(h, p, w3, pos=pos_arg, final=final)
        fasts.append(fast)
        slows.append(slow)

    logits = _logits_call(h.reshape(B * S, H), params['tok_emb'])
    return logits.reshape(B, S, V), jnp.stack(fasts), jnp.stack(slows)


# speedup vs baseline: 1.1848x; 1.0178x over previous
import functools

import jax
import jax.numpy as jnp
import numpy as np
from jax import lax
from jax.experimental import pallas as pl
from jax.experimental.pallas import tpu as pltpu
from jax.experimental.pallas import tpu_sc as plsc

B, S, H, V, HCM, SL, L, I = 2, 2048, 768, 8192, 256, 32, 2, 3072

TS = 1024
LT_R = 1024
LT_V = 2048


def _decay_weights():
    t = np.arange(S)
    n_flags = ((S - 1) // 10) + 1
    m = np.ceil(t / 10).astype(np.int64)
    pow9 = np.ones(n_flags + 1, dtype=np.float64)
    for k in range(1, n_flags + 1):
        pow9[k] = pow9[k - 1] * 0.9
    w = pow9[n_flags - m].astype(np.float32)
    return jnp.asarray(w.reshape(S // TS, 1, TS))


def _sc_embed_gather(table, idx_flat):
    info = plsc.get_sparse_core_info()
    nc, ns = info.num_cores, info.num_subcores
    nw = nc * ns
    rows = idx_flat.shape[0]
    bpw = rows // nw
    mesh = plsc.VectorSubcoreMesh(core_axis_name="c", subcore_axis_name="s")

    @functools.partial(
        pl.kernel,
        mesh=mesh,
        out_type=jax.ShapeDtypeStruct((rows, H), jnp.float32),
        scratch_types=[
            pltpu.VMEM((bpw,), jnp.int32),
            pltpu.VMEM((bpw, H), jnp.float32),
            pltpu.SemaphoreType.DMA,
        ],
    )
    def gather_k(table_hbm, idx_hbm, out_hbm, idx_v, rows_v, sem):
        wid = lax.axis_index("s") * nc + lax.axis_index("c")
        base = wid * bpw
        pltpu.sync_copy(idx_hbm.at[pl.ds(base, bpw)], idx_v)
        pltpu.async_copy(table_hbm.at[idx_v], rows_v, sem).wait()
        pltpu.sync_copy(rows_v, out_hbm.at[pl.ds(base, bpw)])

    return gather_k(table, idx_flat)


def _ln(x, g, b):
    m = x.mean(-1, keepdims=True)
    v = ((x - m) ** 2).mean(-1, keepdims=True)
    return (x - m) / jnp.sqrt(v + 1e-5) * g + b


def _dot(a, b):
    return jnp.dot(a, b, preferred_element_type=jnp.float32)


def _make_layer_body(add_pos, final_ln):
    def body(*refs):
        it = iter(refs)
        h_ref = next(it)
        pos_ref = next(it) if add_pos else None
        w_ref = next(it)
        wi, bi = next(it), next(it)
        wg, bg = next(it), next(it)
        wq, bq = next(it), next(it)
        wo, bo = next(it), next(it)
        ln_g, ln_b = next(it), next(it)
        f1, fb1 = next(it), next(it)
        f2, fb2 = next(it), next(it)
        fln_g, fln_b = next(it), next(it)
        if final_ln:
            fin_g, fin_b = next(it), next(it)
        h_out = next(it)
        fast_out = next(it)
        slow_out = next(it)
        wqo = next(it)
        fbias = next(it)

        b = pl.program_id(0)
        t = pl.program_id(1)

        @pl.when(jnp.logical_and(b == 0, t == 0))
        def _():
            wqo[...] = _dot(wq[...], wo[...])
            fbias[...] = _dot(bq[...], wo[...]) + bo[...]

        h = h_ref[0]
        if add_pos:
            h = h + pos_ref[...]

        fg = jax.nn.sigmoid(_dot(h, wg[...]) + bg[...])
        wv = w_ref[0]
        wfg = fg * wv.reshape(TS, 1)
        dn = (((0,), (0,)), ((), ()))
        pre_f = lax.dot_general(wfg, h, dn,
                                preferred_element_type=jnp.float32)
        pre_s = lax.dot_general(fg - wfg, h, dn,
                                preferred_element_type=jnp.float32)
        sum_f = jnp.sum(wfg, axis=0)
        sum_s = jnp.sum(fg, axis=0) - sum_f
        fa = _dot(pre_f, wi[...]) + sum_f[:, None] * bi[...]
        sa = _dot(pre_s, wi[...]) + sum_s[:, None] * bi[...]

        @pl.when(t == 0)
        def _():
            fast_out[0] = fa
            slow_out[0] = sa

        @pl.when(t != 0)
        def _():
            fast_out[0] += fa
            slow_out[0] += sa

        out = _dot(h, wqo[...]) + fbias[...]
        h1 = _ln(h + out, ln_g[...], ln_b[...])
        ffn = fb2[...]
        half = I // 2
        for c in range(2):
            cs = pl.ds(c * half, half)
            act = jax.nn.gelu(_dot(h1, f1[:, cs]) + fb1[:, cs])
            ffn = ffn + _dot(act, f2[cs, :])
        h2 = _ln(h1 + ffn, fln_g[...], fln_b[...])
        if final_ln:
            h_out[0] = _ln(h2, fin_g[...], fin_b[...])
        else:
            h_out[0] = h2

    return body


def _layer_call(h, p, w3, pos=None, final=None):
    add_pos = pos is not None
    final_ln = final is not None
    row = lambda x: x.reshape(1, -1)
    full2 = lambda a: pl.BlockSpec(a.shape, lambda b, t: (0, 0))

    inputs = [h]
    in_specs = [pl.BlockSpec((1, TS, H), lambda b, t: (b, t, 0))]
    if add_pos:
        inputs.append(pos)
        in_specs.append(pl.BlockSpec((TS, H), lambda b, t: (t, 0)))
    inputs.append(w3)
    in_specs.append(pl.BlockSpec((1, 1, TS), lambda b, t: (t, 0, 0)))

    wmats = [
        p['Wi'], row(p['bi']),
        p['Wg'][:H, :SL], row(p['bg'][:SL]),
        p['Wq'], row(p['bq']),
        p['Wo'][:H], row(p['bo']),
        row(p['ln_g']), row(p['ln_b']),
        p['F1'], row(p['fb1']),
        p['F2'], row(p['fb2']),
        row(p['fln_g']), row(p['fln_b']),
    ]
    if final_ln:
        wmats += [row(final[0]), row(final[1])]
    inputs += wmats
    in_specs += [full2(a) for a in wmats]

    out_shape = [
        jax.ShapeDtypeStruct((B, S, H), jnp.float32),
        jax.ShapeDtypeStruct((B, SL, HCM), jnp.float32),
        jax.ShapeDtypeStruct((B, SL, HCM), jnp.float32),
    ]
    out_specs = [
        pl.BlockSpec((1, TS, H), lambda b, t: (b, t, 0)),
        pl.BlockSpec((1, SL, HCM), lambda b, t: (b, 0, 0)),
        pl.BlockSpec((1, SL, HCM), lambda b, t: (b, 0, 0)),
    ]

    return pl.pallas_call(
        _make_layer_body(add_pos, final_ln),
        grid=(B, S // TS),
        in_specs=in_specs,
        out_specs=out_specs,
        out_shape=out_shape,
        scratch_shapes=[
            pltpu.VMEM((H, H), jnp.float32),
            pltpu.VMEM((1, H), jnp.float32),
        ],
    )(*inputs)


def _logits_body(h_ref, emb_ref, out_ref):
    out_ref[...] = lax.dot_general(
        h_ref[...], emb_ref[...], (((1,), (1,)), ((), ())),
        preferred_element_type=jnp.float32)


def _logits_call(hln_flat, tok_emb):
    rows = hln_flat.shape[0]
    return pl.pallas_call(
        _logits_body,
        grid=(V // LT_V, rows // LT_R),
        in_specs=[
            pl.BlockSpec((LT_R, H), lambda v, r: (r, 0)),
            pl.BlockSpec((LT_V, H), lambda v, r: (v, 0)),
        ],
        out_specs=pl.BlockSpec((LT_R, LT_V), lambda v, r: (r, v)),
        out_shape=jax.ShapeDtypeStruct((rows, V), jnp.float32),
    )(hln_flat, tok_emb)


def kernel(input_ids, params):
    ids_flat = input_ids.reshape(-1).astype(jnp.int32)
    emb = _sc_embed_gather(params['tok_emb'], ids_flat)
    h = emb.reshape(B, S, H)
    w3 = _decay_weights()
    pos = params['pos_emb'][:S]

    layers = params['layers']
    fasts, slows = [], []
    for li, p in enumerate(layers):
        pos_arg = pos if li == 0 else None
        final = (params['final_g'], params['final_b']) if li == L - 1 else None
        h, fast, slow = _layer_call(h, p, w3, pos=pos_arg, final=final)
        fasts.append(fast)
        slows.append(slow)

    logits = _logits_call(h.reshape(B * S, H), params['tok_emb'])
    return logits.reshape(B, S, V), jnp.stack(fasts), jnp.stack(slows)


# speedup vs baseline: 1.1901x; 1.0044x over previous
import functools

import jax
import jax.numpy as jnp
import numpy as np
from jax import lax
from jax.experimental import pallas as pl
from jax.experimental.pallas import tpu as pltpu
from jax.experimental.pallas import tpu_sc as plsc

B, S, H, V, HCM, SL, L, I = 2, 2048, 768, 8192, 256, 32, 2, 3072

TS = 512
HTS = 256
LT_R = 1024
LT_V = 2048


def _decay_weights():
    t = np.arange(S)
    n_flags = ((S - 1) // 10) + 1
    m = np.ceil(t / 10).astype(np.int64)
    pow9 = np.ones(n_flags + 1, dtype=np.float64)
    for k in range(1, n_flags + 1):
        pow9[k] = pow9[k - 1] * 0.9
    w = pow9[n_flags - m].astype(np.float32)
    return jnp.asarray(w.reshape(S // TS, 1, TS))


def _sc_embed_gather(table, idx_flat):
    info = plsc.get_sparse_core_info()
    nc, ns = info.num_cores, info.num_subcores
    nw = nc * ns
    rows = idx_flat.shape[0]
    bpw = rows // nw
    mesh = plsc.VectorSubcoreMesh(core_axis_name="c", subcore_axis_name="s")

    @functools.partial(
        pl.kernel,
        mesh=mesh,
        out_type=jax.ShapeDtypeStruct((rows, H), jnp.float32),
        scratch_types=[
            pltpu.VMEM((bpw,), jnp.int32),
            pltpu.VMEM((bpw, H), jnp.float32),
            pltpu.SemaphoreType.DMA,
        ],
    )
    def gather_k(table_hbm, idx_hbm, out_hbm, idx_v, rows_v, sem):
        wid = lax.axis_index("s") * nc + lax.axis_index("c")
        base = wid * bpw
        pltpu.sync_copy(idx_hbm.at[pl.ds(base, bpw)], idx_v)
        pltpu.async_copy(table_hbm.at[idx_v], rows_v, sem).wait()
        pltpu.sync_copy(rows_v, out_hbm.at[pl.ds(base, bpw)])

    return gather_k(table, idx_flat)


def _ln(x, g, b):
    m = x.mean(-1, keepdims=True)
    v = ((x - m) ** 2).mean(-1, keepdims=True)
    return (x - m) / jnp.sqrt(v + 1e-5) * g + b


def _dot(a, b):
    return jnp.dot(a, b, preferred_element_type=jnp.float32)


def _make_layer_body(add_pos, final_ln):
    def body(*refs):
        it = iter(refs)
        h_ref = next(it)
        pos_ref = next(it) if add_pos else None
        w_ref = next(it)
        wi, bi = next(it), next(it)
        wg, bg = next(it), next(it)
        wq, bq = next(it), next(it)
        wo, bo = next(it), next(it)
        ln_g, ln_b = next(it), next(it)
        f1, fb1 = next(it), next(it)
        f2, fb2 = next(it), next(it)
        fln_g, fln_b = next(it), next(it)
        if final_ln:
            fin_g, fin_b = next(it), next(it)
        h_out = next(it)
        fast_out = next(it)
        slow_out = next(it)
        wqo = next(it)
        fbias = next(it)

        b = pl.program_id(0)
        t = pl.program_id(1)

        @pl.when(jnp.logical_and(b == 0, t == 0))
        def _():
            wqo[...] = _dot(wq[...], wo[...])
            fbias[...] = _dot(bq[...], wo[...]) + bo[...]

        h = h_ref[0]
        if add_pos:
            h = h + pos_ref[...]
        wv = w_ref[0].reshape(TS, 1)
        dn = (((0,), (0,)), ((), ()))
        ihalf = I // 3
        NH = TS // HTS

        fa = sa = None
        h2s = []
        for c in range(NH):
            hh = h[c * HTS:(c + 1) * HTS]
            wvh = wv[c * HTS:(c + 1) * HTS]

            fg = jax.nn.sigmoid(_dot(hh, wg[...]) + bg[...])
            wfg = fg * wvh
            pre_f = lax.dot_general(wfg, hh, dn,
                                    preferred_element_type=jnp.float32)
            pre_s = lax.dot_general(fg - wfg, hh, dn,
                                    preferred_element_type=jnp.float32)
            sum_f = jnp.sum(wfg, axis=0)
            sum_s = jnp.sum(fg, axis=0) - sum_f
            fac = _dot(pre_f, wi[...]) + sum_f[:, None] * bi[...]
            sac = _dot(pre_s, wi[...]) + sum_s[:, None] * bi[...]
            fa = fac if fa is None else fa + fac
            sa = sac if sa is None else sa + sac

            out = _dot(hh, wqo[...]) + fbias[...]
            h1 = _ln(hh + out, ln_g[...], ln_b[...])
            ffn = fb2[...]
            for ci in range(3):
                cs = pl.ds(ci * ihalf, ihalf)
                act = jax.nn.gelu(_dot(h1, f1[:, cs]) + fb1[:, cs])
                ffn = ffn + _dot(act, f2[cs, :])
            h2 = _ln(h1 + ffn, fln_g[...], fln_b[...])
            if final_ln:
                h2 = _ln(h2, fin_g[...], fin_b[...])
            h2s.append(h2)

        for c in range(NH):
            h_out[0, c * HTS:(c + 1) * HTS] = h2s[c]

        @pl.when(t == 0)
        def _():
            fast_out[0] = fa
            slow_out[0] = sa

        @pl.when(t != 0)
        def _():
            fast_out[0] += fa
            slow_out[0] += sa

    return body


def _layer_call(h, p, w3, pos=None, final=None):
    add_pos = pos is not None
    final_ln = final is not None
    row = lambda x: x.reshape(1, -1)
    full2 = lambda a: pl.BlockSpec(a.shape, lambda b, t: (0, 0))

    inputs = [h]
    in_specs = [pl.BlockSpec((1, TS, H), lambda b, t: (b, t, 0))]
    if add_pos:
        inputs.append(pos)
        in_specs.append(pl.BlockSpec((TS, H), lambda b, t: (t, 0)))
    inputs.append(w3)
    in_specs.append(pl.BlockSpec((1, 1, TS), lambda b, t: (t, 0, 0)))

    wmats = [
        p['Wi'], row(p['bi']),
        p['Wg'][:H, :SL], row(p['bg'][:SL]),
        p['Wq'], row(p['bq']),
        p['Wo'][:H], row(p['bo']),
        row(p['ln_g']), row(p['ln_b']),
        p['F1'], row(p['fb1']),
        p['F2'], row(p['fb2']),
        row(p['fln_g']), row(p['fln_b']),
    ]
    if final_ln:
        wmats += [row(final[0]), row(final[1])]
    inputs += wmats
    in_specs += [full2(a) for a in wmats]

    out_shape = [
        jax.ShapeDtypeStruct((B, S, H), jnp.float32),
        jax.ShapeDtypeStruct((B, SL, HCM), jnp.float32),
        jax.ShapeDtypeStruct((B, SL, HCM), jnp.float32),
    ]
    out_specs = [
        pl.BlockSpec((1, TS, H), lambda b, t: (b, t, 0)),
        pl.BlockSpec((1, SL, HCM), lambda b, t: (b, 0, 0)),
        pl.BlockSpec((1, SL, HCM), lambda b, t: (b, 0, 0)),
    ]

    return pl.pallas_call(
        _make_layer_body(add_pos, final_ln),
        grid=(B, S // TS),
        in_specs=in_specs,
        out_specs=out_specs,
        out_shape=out_shape,
        scratch_shapes=[
            pltpu.VMEM((H, H), jnp.float32),
            pltpu.VMEM((1, H), jnp.float32),
        ],
    )(*inputs)


def _logits_body(h_ref, emb_ref, out_ref):
    out_ref[...] = lax.dot_general(
        h_ref[...], emb_ref[...], (((1,), (1,)), ((), ())),
        preferred_element_type=jnp.float32)


def _logits_call(hln_flat, tok_emb):
    rows = hln_flat.shape[0]
    return pl.pallas_call(
        _logits_body,
        grid=(V // LT_V, rows // LT_R),
        in_specs=[
            pl.BlockSpec((LT_R, H), lambda v, r: (r, 0)),
            pl.BlockSpec((LT_V, H), lambda v, r: (v, 0)),
        ],
        out_specs=pl.BlockSpec((LT_R, LT_V), lambda v, r: (r, v)),
        out_shape=jax.ShapeDtypeStruct((rows, V), jnp.float32),
    )(hln_flat, tok_emb)


def kernel(input_ids, params):
    ids_flat = input_ids.reshape(-1).astype(jnp.int32)
    emb = _sc_embed_gather(params['tok_emb'], ids_flat)
    h = emb.reshape(B, S, H)
    w3 = _decay_weights()
    pos = params['pos_emb'][:S]

    layers = params['layers']
    fasts, slows = [], []
    for li, p in enumerate(layers):
        pos_arg = pos if li == 0 else None
        final = (params['final_g'], params['final_b']) if li == L - 1 else None
        h, fast, slow = _layer_call(h, p, w3, pos=pos_arg, final=final)
        fasts.append(fast)
        slows.append(slow)

    logits = _logits_call(h.reshape(B * S, H), params['tok_emb'])
    return logits.reshape(B, S, V), jnp.stack(fasts), jnp.stack(slows)


# speedup vs baseline: 1.2221x; 1.0270x over previous
import functools

import jax
import jax.numpy as jnp
import numpy as np
from jax import lax
from jax.experimental import pallas as pl
from jax.experimental.pallas import tpu as pltpu
from jax.experimental.pallas import tpu_sc as plsc

B, S, H, V, HCM, SL, L, I = 2, 2048, 768, 8192, 256, 32, 2, 3072

TS = 1024
HTS = 512
LT_R = 2048
LT_V = 2048
_VMEM_LIMIT = 66 * 1024 * 1024


def _decay_weights():
    t = np.arange(S)
    n_flags = ((S - 1) // 10) + 1
    m = np.ceil(t / 10).astype(np.int64)
    pow9 = np.ones(n_flags + 1, dtype=np.float64)
    for k in range(1, n_flags + 1):
        pow9[k] = pow9[k - 1] * 0.9
    w = pow9[n_flags - m].astype(np.float32)
    return jnp.asarray(w.reshape(S // TS, 1, TS))


def _sc_embed_gather(table, idx_flat):
    info = plsc.get_sparse_core_info()
    nc, ns = info.num_cores, info.num_subcores
    nw = nc * ns
    rows = idx_flat.shape[0]
    bpw = rows // nw
    mesh = plsc.VectorSubcoreMesh(core_axis_name="c", subcore_axis_name="s")

    @functools.partial(
        pl.kernel,
        mesh=mesh,
        out_type=jax.ShapeDtypeStruct((rows, H), jnp.float32),
        scratch_types=[
            pltpu.VMEM((bpw,), jnp.int32),
            pltpu.VMEM((bpw, H), jnp.float32),
            pltpu.SemaphoreType.DMA,
        ],
    )
    def gather_k(table_hbm, idx_hbm, out_hbm, idx_v, rows_v, sem):
        wid = lax.axis_index("s") * nc + lax.axis_index("c")
        base = wid * bpw
        pltpu.sync_copy(idx_hbm.at[pl.ds(base, bpw)], idx_v)
        pltpu.async_copy(table_hbm.at[idx_v], rows_v, sem).wait()
        pltpu.sync_copy(rows_v, out_hbm.at[pl.ds(base, bpw)])

    return gather_k(table, idx_flat)


def _ln(x, g, b):
    m = x.mean(-1, keepdims=True)
    v = ((x - m) ** 2).mean(-1, keepdims=True)
    return (x - m) / jnp.sqrt(v + 1e-5) * g + b


def _dot(a, b):
    return jnp.dot(a, b, preferred_element_type=jnp.float32)


def _make_layer_body(add_pos, final_ln):
    def body(*refs):
        it = iter(refs)
        h_ref = next(it)
        pos_ref = next(it) if add_pos else None
        w_ref = next(it)
        wi, bi = next(it), next(it)
        wg, bg = next(it), next(it)
        wq, bq = next(it), next(it)
        wo, bo = next(it), next(it)
        ln_g, ln_b = next(it), next(it)
        f1, fb1 = next(it), next(it)
        f2, fb2 = next(it), next(it)
        fln_g, fln_b = next(it), next(it)
        if final_ln:
            fin_g, fin_b = next(it), next(it)
        h_out = next(it)
        fast_out = next(it)
        slow_out = next(it)
        wqo = next(it)
        fbias = next(it)

        b = pl.program_id(0)
        t = pl.program_id(1)

        @pl.when(jnp.logical_and(b == 0, t == 0))
        def _():
            wqo[...] = _dot(wq[...], wo[...])
            fbias[...] = _dot(bq[...], wo[...]) + bo[...]

        h = h_ref[0]
        if add_pos:
            h = h + pos_ref[...]
        wv = w_ref[0].reshape(TS, 1)
        dn = (((0,), (0,)), ((), ()))
        ihalf = I // 3
        NH = TS // HTS

        fa = sa = None
        h2s = []
        for c in range(NH):
            hh = h[c * HTS:(c + 1) * HTS]
            wvh = wv[c * HTS:(c + 1) * HTS]

            fg = jax.nn.sigmoid(_dot(hh, wg[...]) + bg[...])
            wfg = fg * wvh
            pre_f = lax.dot_general(wfg, hh, dn,
                                    preferred_element_type=jnp.float32)
            pre_s = lax.dot_general(fg - wfg, hh, dn,
                                    preferred_element_type=jnp.float32)
            sum_f = jnp.sum(wfg, axis=0)
            sum_s = jnp.sum(fg, axis=0) - sum_f
            fac = _dot(pre_f, wi[...]) + sum_f[:, None] * bi[...]
            sac = _dot(pre_s, wi[...]) + sum_s[:, None] * bi[...]
            fa = fac if fa is None else fa + fac
            sa = sac if sa is None else sa + sac

            out = _dot(hh, wqo[...]) + fbias[...]
            h1 = _ln(hh + out, ln_g[...], ln_b[...])
            ffn = fb2[...]
            for ci in range(3):
                cs = pl.ds(ci * ihalf, ihalf)
                act = jax.nn.gelu(_dot(h1, f1[:, cs]) + fb1[:, cs])
                ffn = ffn + _dot(act, f2[cs, :])
            h2 = _ln(h1 + ffn, fln_g[...], fln_b[...])
            if final_ln:
                h2 = _ln(h2, fin_g[...], fin_b[...])
            h2s.append(h2)

        for c in range(NH):
            h_out[0, c * HTS:(c + 1) * HTS] = h2s[c]

        @pl.when(t == 0)
        def _():
            fast_out[0] = fa
            slow_out[0] = sa

        @pl.when(t != 0)
        def _():
            fast_out[0] += fa
            slow_out[0] += sa

    return body


def _layer_call(h, p, w3, pos=None, final=None):
    add_pos = pos is not None
    final_ln = final is not None
    row = lambda x: x.reshape(1, -1)
    full2 = lambda a: pl.BlockSpec(a.shape, lambda b, t: (0, 0))

    inputs = [h]
    in_specs = [pl.BlockSpec((1, TS, H), lambda b, t: (b, t, 0))]
    if add_pos:
        inputs.append(pos)
        in_specs.append(pl.BlockSpec((TS, H), lambda b, t: (t, 0)))
    inputs.append(w3)
    in_specs.append(pl.BlockSpec((1, 1, TS), lambda b, t: (t, 0, 0)))

    wmats = [
        p['Wi'], row(p['bi']),
        p['Wg'][:H, :SL], row(p['bg'][:SL]),
        p['Wq'], row(p['bq']),
        p['Wo'][:H], row(p['bo']),
        row(p['ln_g']), row(p['ln_b']),
        p['F1'], row(p['fb1']),
        p['F2'], row(p['fb2']),
        row(p['fln_g']), row(p['fln_b']),
    ]
    if final_ln:
        wmats += [row(final[0]), row(final[1])]
    inputs += wmats
    in_specs += [full2(a) for a in wmats]

    out_shape = [
        jax.ShapeDtypeStruct((B, S, H), jnp.float32),
        jax.ShapeDtypeStruct((B, SL, HCM), jnp.float32),
        jax.ShapeDtypeStruct((B, SL, HCM), jnp.float32),
    ]
    out_specs = [
        pl.BlockSpec((1, TS, H), lambda b, t: (b, t, 0)),
        pl.BlockSpec((1, SL, HCM), lambda b, t: (b, 0, 0)),
        pl.BlockSpec((1, SL, HCM), lambda b, t: (b, 0, 0)),
    ]

    return pl.pallas_call(
        _make_layer_body(add_pos, final_ln),
        grid=(B, S // TS),
        in_specs=in_specs,
        out_specs=out_specs,
        out_shape=out_shape,
        scratch_shapes=[
            pltpu.VMEM((H, H), jnp.float32),
            pltpu.VMEM((1, H), jnp.float32),
        ],
        compiler_params=pltpu.CompilerParams(vmem_limit_bytes=_VMEM_LIMIT),
    )(*inputs)


def _logits_body(h_ref, emb_ref, out_ref):
    out_ref[...] = lax.dot_general(
        h_ref[...], emb_ref[...], (((1,), (1,)), ((), ())),
        preferred_element_type=jnp.float32)


def _logits_call(hln_flat, tok_emb):
    rows = hln_flat.shape[0]
    return pl.pallas_call(
        _logits_body,
        grid=(V // LT_V, rows // LT_R),
        in_specs=[
            pl.BlockSpec((LT_R, H), lambda v, r: (r, 0)),
            pl.BlockSpec((LT_V, H), lambda v, r: (v, 0)),
        ],
        out_specs=pl.BlockSpec((LT_R, LT_V), lambda v, r: (r, v)),
        out_shape=jax.ShapeDtypeStruct((rows, V), jnp.float32),
        compiler_params=pltpu.CompilerParams(vmem_limit_bytes=_VMEM_LIMIT),
    )(hln_flat, tok_emb)


def kernel(input_ids, params):
    ids_flat = input_ids.reshape(-1).astype(jnp.int32)
    emb = _sc_embed_gather(params['tok_emb'], ids_flat)
    h = emb.reshape(B, S, H)
    w3 = _decay_weights()
    pos = params['pos_emb'][:S]

    layers = params['layers']
    fasts, slows = [], []
    for li, p in enumerate(layers):
        pos_arg = pos if li == 0 else None
        final = (params['final_g'], params['final_b']) if li == L - 1 else None
        h, fast, slow = _layer_call(h, p, w3, pos=pos_arg, final=final)
        fasts.append(fast)
        slows.append(slow)

    logits = _logits_call(h.reshape(B * S, H), params['tok_emb'])
    return logits.reshape(B, S, V), jnp.stack(fasts), jnp.stack(slows)


# speedup vs baseline: 1.2283x; 1.0050x over previous
import functools

import jax
import jax.numpy as jnp
import numpy as np
from jax import lax
from jax.experimental import pallas as pl
from jax.experimental.pallas import tpu as pltpu
from jax.experimental.pallas import tpu_sc as plsc

B, S, H, V, HCM, SL, L, I = 2, 2048, 768, 8192, 256, 32, 2, 3072

TS = 1024
HTS = 512
LT_R = 2048
LT_V = 2048
_VMEM_LIMIT = 100 * 1024 * 1024


def _decay_weights():
    t = np.arange(S)
    n_flags = ((S - 1) // 10) + 1
    m = np.ceil(t / 10).astype(np.int64)
    pow9 = np.ones(n_flags + 1, dtype=np.float64)
    for k in range(1, n_flags + 1):
        pow9[k] = pow9[k - 1] * 0.9
    w = pow9[n_flags - m].astype(np.float32)
    return jnp.asarray(w.reshape(S // TS, 1, TS))


def _sc_embed_gather(table, idx_flat):
    info = plsc.get_sparse_core_info()
    nc, ns = info.num_cores, info.num_subcores
    nw = nc * ns
    rows = idx_flat.shape[0]
    bpw = rows // nw
    mesh = plsc.VectorSubcoreMesh(core_axis_name="c", subcore_axis_name="s")

    @functools.partial(
        pl.kernel,
        mesh=mesh,
        out_type=jax.ShapeDtypeStruct((rows, H), jnp.float32),
        scratch_types=[
            pltpu.VMEM((bpw,), jnp.int32),
            pltpu.VMEM((bpw, H), jnp.float32),
            pltpu.SemaphoreType.DMA,
        ],
    )
    def gather_k(table_hbm, idx_hbm, out_hbm, idx_v, rows_v, sem):
        wid = lax.axis_index("s") * nc + lax.axis_index("c")
        base = wid * bpw
        pltpu.sync_copy(idx_hbm.at[pl.ds(base, bpw)], idx_v)
        pltpu.async_copy(table_hbm.at[idx_v], rows_v, sem).wait()
        pltpu.sync_copy(rows_v, out_hbm.at[pl.ds(base, bpw)])

    return gather_k(table, idx_flat)


def _ln(x, g, b):
    m = x.mean(-1, keepdims=True)
    v = ((x - m) ** 2).mean(-1, keepdims=True)
    return (x - m) / jnp.sqrt(v + 1e-5) * g + b


def _dot(a, b):
    return jnp.dot(a, b, preferred_element_type=jnp.float32)


def _make_layer_body(add_pos, final_ln):
    def body(*refs):
        it = iter(refs)
        h_ref = next(it)
        pos_ref = next(it) if add_pos else None
        w_ref = next(it)
        wi, bi = next(it), next(it)
        wg, bg = next(it), next(it)
        wq, bq = next(it), next(it)
        wo, bo = next(it), next(it)
        ln_g, ln_b = next(it), next(it)
        f1, fb1 = next(it), next(it)
        f2, fb2 = next(it), next(it)
        fln_g, fln_b = next(it), next(it)
        if final_ln:
            fin_g, fin_b = next(it), next(it)
        h_out = next(it)
        fast_out = next(it)
        slow_out = next(it)
        wqo = next(it)
        fbias = next(it)

        b = pl.program_id(0)
        t = pl.program_id(1)

        @pl.when(jnp.logical_and(b == 0, t == 0))
        def _():
            wqo[...] = _dot(wq[...], wo[...])
            fbias[...] = _dot(bq[...], wo[...]) + bo[...]

        h = h_ref[0]
        if add_pos:
            h = h + pos_ref[...]
        wv = w_ref[0].reshape(TS, 1)
        dn = (((0,), (0,)), ((), ()))
        ihalf = I // 3
        NH = TS // HTS

        fa = sa = None
        h2s = []
        for c in range(NH):
            hh = h[c * HTS:(c + 1) * HTS]
            wvh = wv[c * HTS:(c + 1) * HTS]

            fg = jax.nn.sigmoid(_dot(hh, wg[...]) + bg[...])
            wfg = fg * wvh
            pre_f = lax.dot_general(wfg, hh, dn,
                                    preferred_element_type=jnp.float32)
            pre_s = lax.dot_general(fg - wfg, hh, dn,
                                    preferred_element_type=jnp.float32)
            sum_f = jnp.sum(wfg, axis=0)
            sum_s = jnp.sum(fg, axis=0) - sum_f
            fac = _dot(pre_f, wi[...]) + sum_f[:, None] * bi[...]
            sac = _dot(pre_s, wi[...]) + sum_s[:, None] * bi[...]
            fa = fac if fa is None else fa + fac
            sa = sac if sa is None else sa + sac

            out = _dot(hh, wqo[...]) + fbias[...]
            h1 = _ln(hh + out, ln_g[...], ln_b[...])
            ffn = fb2[...]
            for ci in range(3):
                cs = pl.ds(ci * ihalf, ihalf)
                act = jax.nn.gelu(_dot(h1, f1[:, cs]) + fb1[:, cs])
                ffn = ffn + _dot(act, f2[cs, :])
            h2 = _ln(h1 + ffn, fln_g[...], fln_b[...])
            if final_ln:
                h2 = _ln(h2, fin_g[...], fin_b[...])
            h2s.append(h2)

        for c in range(NH):
            h_out[0, c * HTS:(c + 1) * HTS] = h2s[c]

        @pl.when(t == 0)
        def _():
            fast_out[0] = fa
            slow_out[0] = sa

        @pl.when(t != 0)
        def _():
            fast_out[0] += fa
            slow_out[0] += sa

    return body


def _layer_call(h, p, w3, pos=None, final=None):
    add_pos = pos is not None
    final_ln = final is not None
    row = lambda x: x.reshape(1, -1)
    full2 = lambda a: pl.BlockSpec(a.shape, lambda b, t: (0, 0))

    inputs = [h]
    in_specs = [pl.BlockSpec((1, TS, H), lambda b, t: (b, t, 0))]
    if add_pos:
        inputs.append(pos)
        in_specs.append(pl.BlockSpec((TS, H), lambda b, t: (t, 0)))
    inputs.append(w3)
    in_specs.append(pl.BlockSpec((1, 1, TS), lambda b, t: (t, 0, 0)))

    wmats = [
        p['Wi'], row(p['bi']),
        p['Wg'][:H, :SL], row(p['bg'][:SL]),
        p['Wq'], row(p['bq']),
        p['Wo'][:H], row(p['bo']),
        row(p['ln_g']), row(p['ln_b']),
        p['F1'], row(p['fb1']),
        p['F2'], row(p['fb2']),
        row(p['fln_g']), row(p['fln_b']),
    ]
    if final_ln:
        wmats += [row(final[0]), row(final[1])]
    inputs += wmats
    in_specs += [full2(a) for a in wmats]

    out_shape = [
        jax.ShapeDtypeStruct((B, S, H), jnp.float32),
        jax.ShapeDtypeStruct((B, SL, HCM), jnp.float32),
        jax.ShapeDtypeStruct((B, SL, HCM), jnp.float32),
    ]
    out_specs = [
        pl.BlockSpec((1, TS, H), lambda b, t: (b, t, 0)),
        pl.BlockSpec((1, SL, HCM), lambda b, t: (b, 0, 0)),
        pl.BlockSpec((1, SL, HCM), lambda b, t: (b, 0, 0)),
    ]

    return pl.pallas_call(
        _make_layer_body(add_pos, final_ln),
        grid=(B, S // TS),
        in_specs=in_specs,
        out_specs=out_specs,
        out_shape=out_shape,
        scratch_shapes=[
            pltpu.VMEM((H, H), jnp.float32),
            pltpu.VMEM((1, H), jnp.float32),
        ],
        compiler_params=pltpu.CompilerParams(vmem_limit_bytes=_VMEM_LIMIT),
    )(*inputs)


def _logits_body(h_ref, emb_ref, out_ref):
    out_ref[...] = lax.dot_general(
        h_ref[...], emb_ref[...], (((1,), (1,)), ((), ())),
        preferred_element_type=jnp.float32)


def _logits_call(hln_flat, tok_emb):
    rows = hln_flat.shape[0]
    return pl.pallas_call(
        _logits_body,
        grid=(V // LT_V, rows // LT_R),
        in_specs=[
            pl.BlockSpec((LT_R, H), lambda v, r: (r, 0)),
            pl.BlockSpec((LT_V, H), lambda v, r: (v, 0)),
        ],
        out_specs=pl.BlockSpec((LT_R, LT_V), lambda v, r: (r, v)),
        out_shape=jax.ShapeDtypeStruct((rows, V), jnp.float32),
        compiler_params=pltpu.CompilerParams(vmem_limit_bytes=_VMEM_LIMIT),
    )(hln_flat, tok_emb)


def kernel(input_ids, params):
    ids_flat = input_ids.reshape(-1).astype(jnp.int32)
    emb = _sc_embed_gather(params['tok_emb'], ids_flat)
    h = emb.reshape(B, S, H)
    w3 = _decay_weights()
    pos = params['pos_emb'][:S]

    layers = params['layers']
    fasts, slows = [], []
    for li, p in enumerate(layers):
        pos_arg = pos if li == 0 else None
        final = (params['final_g'], params['final_b']) if li == L - 1 else None
        h, fast, slow = _layer_call(h, p, w3, pos=pos_arg, final=final)
        fasts.append(fast)
        slows.append(slow)

    logits = _logits_call(h.reshape(B * S, H), params['tok_emb'])
    return logits.reshape(B, S, V), jnp.stack(fasts), jnp.stack(slows)
